# Initial kernel scaffold; baseline (speedup 1.0000x reference)
import jax
import jax.numpy as jnp
from jax.experimental import pallas as pl


def kernel(x_one, edge_index_one, x_two, edge_index_two, W1, b1, W2, b2):
    raise NotImplementedError("write your pallas kernel here")


# speedup vs baseline: 2.0673x; 2.0673x over previous
import jax
import jax.numpy as jnp
from jax import lax
from jax.experimental import pallas as pl
from jax.experimental.pallas import tpu as pltpu
from jax.experimental.pallas import tpu_sc as plsc

N = 10000
NPAD = 10240
D = 128
E = 320000
NC = 2
NS = 16
EPT = E // NS
NV = EPT // 16
RNG = NPAD // NS
LCAP = 1536
LTOT = NS * LCAP
AROW = RNG + 16
GB = 128
NBAT = LCAP // GB
FB = 64
MASK15 = 32767

_mesh = plsc.VectorSubcoreMesh(core_axis_name="c", subcore_axis_name="s")


def _nrsqrt(x):
    i32 = lax.bitcast_convert_type(x, jnp.int32)
    i32 = jnp.int32(0x5F3759DF) - (i32 >> 1)
    y = lax.bitcast_convert_type(i32, jnp.float32)
    for _ in range(3):
        y = y * (1.5 - 0.5 * x * y * y)
    return y


def _scan_body(src_hbm, dst_hbm, lists_hbm, dis_hbm,
               sh, src_v, dst_v, lists_v, hist_v, dis_v, cur):
    c = lax.axis_index("c")
    s = lax.axis_index("s")
    tl = c * NS + s
    zero16i = jnp.zeros((16,), jnp.int32)
    iota = lax.iota(jnp.int32, 16)

    def zl(i, carry):
        lists_v[pl.ds(i * 16, 16)] = zero16i
        return carry

    lax.fori_loop(0, LTOT // 16, zl, 0)

    def zc(o, carry):
        cur[o] = 0
        return carry

    lax.fori_loop(0, NS, zc, 0)
    pltpu.sync_copy(src_hbm.at[pl.ds(c * E + s * EPT, EPT)], src_v)
    pltpu.sync_copy(dst_hbm.at[pl.ds(c * E + s * EPT, EPT)], dst_v)

    srcoff = c * NPAD

    def scan(i, carry):
        dv = dst_v[pl.ds(i * 16, 16)]
        sv = src_v[pl.ds(i * 16, 16)] + srcoff
        ov = ((dv >> 7) * 13108) >> 16
        pk = ((dv - ov * RNG + 16) << 15) | sv
        for l in range(16):
            o = ov[l]
            p = pk[l]
            cu = cur[o]
            base = o * LCAP + (cu & ~jnp.int32(15))
            vec = jnp.where(iota == (cu & 15), p, 0)
            plsc.addupdate(lists_v.at[pl.ds(base, 16)], vec)
            cur[o] = cu + 1
        return carry

    lax.fori_loop(0, NV, scan, 0)
    pltpu.sync_copy(lists_v, lists_hbm.at[pl.ds(tl * LTOT, LTOT)])
    pltpu.sync_copy(lists_v, sh.at[pl.ds(s * LTOT, LTOT)])
    plsc.subcore_barrier()

    for u in range(NS):
        pltpu.sync_copy(
            sh.at[pl.ds(u * LTOT + s * LCAP, LCAP)],
            lists_v.at[pl.ds(u * LCAP, LCAP)],
        )
    zero16f = jnp.zeros((16,), jnp.float32)

    def zh(i, carry):
        hist_v[pl.ds(i * 16, 16)] = zero16f
        return carry

    lax.fori_loop(0, AROW // 16, zh, 0)
    one16f = jnp.full((16,), 1.0, jnp.float32)

    def hist(i, carry):
        pkv = lists_v[pl.ds(i * 16, 16)]
        rf = jnp.minimum(pkv >> 15, AROW - 1)
        for l in range(16):
            r = rf[l]
            base = r & ~jnp.int32(15)
            vec = jnp.where(iota == (r & 15), one16f, zero16f)
            plsc.addupdate(hist_v.at[pl.ds(base, 16)], vec)
        return carry

    lax.fori_loop(0, LTOT // 16, hist, 0)

    def newt(i, carry):
        x = hist_v[pl.ds(i * 16, 16)] + 1.0
        dis_v[pl.ds(i * 16, 16)] = _nrsqrt(x)
        return carry

    lax.fori_loop(0, AROW // 16, newt, 0)
    pltpu.sync_copy(dis_v.at[pl.ds(16, RNG)],
                    dis_hbm.at[pl.ds(c * NPAD + s * RNG, RNG)])


_scan_kernel = pl.kernel(
    _scan_body,
    out_type=(
        jax.ShapeDtypeStruct((NC * NS * LTOT,), jnp.int32),
        jax.ShapeDtypeStruct((NC * NPAD,), jnp.float32),
    ),
    mesh=_mesh,
    scratch_types=[
        pltpu.VMEM_SHARED((NS * LTOT,), jnp.int32),
        pltpu.VMEM((EPT,), jnp.int32),
        pltpu.VMEM((EPT,), jnp.int32),
        pltpu.VMEM((LTOT,), jnp.int32),
        pltpu.VMEM((AROW,), jnp.float32),
        pltpu.VMEM((AROW,), jnp.float32),
        pltpu.SMEM((NS,), jnp.int32),
    ],
)


def _mm_body(x_ref, w_ref, dis_ref, o_ref):
    xw = jnp.dot(x_ref[0], w_ref[0], preferred_element_type=jnp.float32)
    o_ref[0] = xw * dis_ref[0]


_MM_BLK = 640
_mm = pl.pallas_call(
    _mm_body,
    grid=(NC, NPAD // _MM_BLK),
    in_specs=[
        pl.BlockSpec((1, _MM_BLK, D), lambda c, i: (c, i, 0)),
        pl.BlockSpec((1, D, D), lambda c, i: (c, 0, 0)),
        pl.BlockSpec((1, _MM_BLK, 1), lambda c, i: (c, i, 0)),
    ],
    out_specs=pl.BlockSpec((1, _MM_BLK, D), lambda c, i: (c, i, 0)),
    out_shape=jax.ShapeDtypeStruct((NC, NPAD, D), jnp.float32),
)


def _agg_body(xws_hbm, lists_hbm, dis_hbm, b_hbm, out_hbm,
              acc, pk_v, rows0, rows1, idx0, idx1, dis_b, bb, gsem0, gsem1):
    c = lax.axis_index("c")
    s = lax.axis_index("s")
    zero16f = jnp.zeros((16,), jnp.float32)

    def za(i, carry):
        for jj in range(D // 16):
            acc[i, pl.ds(jj * 16, 16)] = zero16f
        return carry

    lax.fori_loop(0, AROW, za, 0)
    pltpu.sync_copy(b_hbm.at[pl.ds(c * D, D)], bb)
    pltpu.sync_copy(dis_hbm.at[pl.ds(c * NPAD + s * RNG, RNG)],
                    dis_b.at[pl.ds(0, RNG)])

    def unpack_idx(m, idxb):
        for mm in range(GB // 16):
            pkv = pk_v[pl.ds(m * GB + mm * 16, 16)]
            sg = jnp.minimum(pkv & MASK15, NC * NPAD - 1)
            idxb[pl.ds(mm * 16, 16)] = sg

    def accum(m, rows):
        for mm in range(GB // 16):
            pkv = pk_v[pl.ds(m * GB + mm * 16, 16)]
            rf = jnp.minimum(pkv >> 15, AROW - 1)
            for l in range(16):
                r = rf[l]
                k = mm * 16 + l
                for jj in range(D // 16):
                    sl = pl.ds(jj * 16, 16)
                    plsc.addupdate(acc.at[r, sl], rows[k, sl])
        return None

    def run_chunk(u, carry):
        pltpu.sync_copy(
            lists_hbm.at[pl.ds((c * NS + u) * LTOT + s * LCAP, LCAP)], pk_v)
        unpack_idx(0, idx0)
        pltpu.async_copy(xws_hbm.at[idx0], rows0, gsem0)

        def pair(i, carry2):
            m = 2 * i
            unpack_idx(m + 1, idx1)
            pltpu.async_copy(xws_hbm.at[idx1], rows1, gsem1)
            pltpu.make_async_copy(xws_hbm.at[idx0], rows0, gsem0).wait()
            accum(m, rows0)

            @pl.when(m + 2 < NBAT)
            def _():
                unpack_idx(m + 2, idx0)
                pltpu.async_copy(xws_hbm.at[idx0], rows0, gsem0)

            pltpu.make_async_copy(xws_hbm.at[idx1], rows1, gsem1).wait()
            accum(m + 1, rows1)
            return carry2

        lax.fori_loop(0, NBAT // 2, pair, 0)
        return carry

    lax.fori_loop(0, NS, run_chunk, 0)

    r0g = c * NPAD + s * RNG

    def fchunk(k, carry):
        pltpu.sync_copy(xws_hbm.at[pl.ds(r0g + k * FB, FB)],
                        rows0.at[pl.ds(0, FB)])

        def frow(r, carry2):
            dsc = dis_b[pl.ds(k * FB + r, 16)]
            dis = dsc[0]
            ar = k * FB + r + 16
            for jj in range(D // 16):
                sl = pl.ds(jj * 16, 16)
                v = dis * (acc[ar, sl] + rows0[r, sl]) + bb[sl]
                rows0[r, sl] = jnp.maximum(v, 0.0)
            return carry2

        lax.fori_loop(0, FB, frow, 0)
        pltpu.sync_copy(rows0.at[pl.ds(0, FB)],
                        out_hbm.at[pl.ds(r0g + k * FB, FB)])
        return carry

    lax.fori_loop(0, RNG // FB, fchunk, 0)


_agg_kernel = pl.kernel(
    _agg_body,
    out_type=jax.ShapeDtypeStruct((NC * NPAD, D), jnp.float32),
    mesh=_mesh,
    scratch_types=[
        pltpu.VMEM((AROW, D), jnp.float32),
        pltpu.VMEM((LCAP,), jnp.int32),
        pltpu.VMEM((GB, D), jnp.float32),
        pltpu.VMEM((GB, D), jnp.float32),
        pltpu.VMEM((GB,), jnp.int32),
        pltpu.VMEM((GB,), jnp.int32),
        pltpu.VMEM((RNG + 16,), jnp.float32),
        pltpu.VMEM((D,), jnp.float32),
        pltpu.SemaphoreType.DMA,
        pltpu.SemaphoreType.DMA,
    ],
)


def kernel(x_one, edge_index_one, x_two, edge_index_two, W1, b1, W2, b2):
    src_all = jnp.concatenate([edge_index_one[0], edge_index_two[0]]).astype(jnp.int32)
    dst_all = jnp.concatenate([edge_index_one[1], edge_index_two[1]]).astype(jnp.int32)

    lists, dis = _scan_kernel(src_all, dst_all)

    x_pad = jnp.pad(jnp.stack([x_one, x_two]), ((0, 0), (0, NPAD - N), (0, 0)))
    W_all = jnp.stack([W1, W2])
    xws = _mm(x_pad, W_all, dis.reshape(NC, NPAD, 1))

    b_all = jnp.concatenate([b1, b2])
    out_flat = _agg_kernel(xws.reshape(NC * NPAD, D), lists, dis, b_all)
    out = out_flat.reshape(NC, NPAD, D)
    return (out[0, :N], out[1, :N])


# speedup vs baseline: 2.0713x; 1.0019x over previous
import jax
import jax.numpy as jnp
from jax import lax
from jax.experimental import pallas as pl
from jax.experimental.pallas import tpu as pltpu
from jax.experimental.pallas import tpu_sc as plsc

N = 10000
NPAD = 10240
D = 128
E = 320000
NC = 2
NS = 16
EPT = E // NS
NV = EPT // 16
RNG = NPAD // NS
LCAP = 1536
LTOT = NS * LCAP
AROW = RNG + 16
GB = 64
NBAT = LCAP // GB
FB = 64
MASK15 = 32767

_mesh = plsc.VectorSubcoreMesh(core_axis_name="c", subcore_axis_name="s")


def _nrsqrt(x):
    i32 = lax.bitcast_convert_type(x, jnp.int32)
    i32 = jnp.int32(0x5F3759DF) - (i32 >> 1)
    y = lax.bitcast_convert_type(i32, jnp.float32)
    for _ in range(3):
        y = y * (1.5 - 0.5 * x * y * y)
    return y


def _scan_body(src_hbm, dst_hbm, lists_hbm, dis_hbm,
               sh, src_v, dst_v, lists_v, hist_v, dis_v, cur):
    c = lax.axis_index("c")
    s = lax.axis_index("s")
    tl = c * NS + s
    zero16i = jnp.zeros((16,), jnp.int32)
    iota = lax.iota(jnp.int32, 16)

    def zl(i, carry):
        lists_v[pl.ds(i * 16, 16)] = zero16i
        return carry

    lax.fori_loop(0, LTOT // 16, zl, 0)

    def zc(o, carry):
        cur[o] = 0
        return carry

    lax.fori_loop(0, NS, zc, 0)
    pltpu.sync_copy(src_hbm.at[pl.ds(c * E + s * EPT, EPT)], src_v)
    pltpu.sync_copy(dst_hbm.at[pl.ds(c * E + s * EPT, EPT)], dst_v)

    srcoff = c * NPAD

    def scan(i, carry):
        dv = dst_v[pl.ds(i * 16, 16)]
        sv = src_v[pl.ds(i * 16, 16)] + srcoff
        ov = ((dv >> 7) * 13108) >> 16
        pk = ((dv - ov * RNG + 16) << 15) | sv
        for l in range(16):
            o = ov[l]
            p = pk[l]
            cu = cur[o]
            base = o * LCAP + (cu & ~jnp.int32(15))
            vec = jnp.where(iota == (cu & 15), p, 0)
            plsc.addupdate(lists_v.at[pl.ds(base, 16)], vec)
            cur[o] = cu + 1
        return carry

    lax.fori_loop(0, NV, scan, 0)
    pltpu.sync_copy(lists_v, lists_hbm.at[pl.ds(tl * LTOT, LTOT)])
    pltpu.sync_copy(lists_v, sh.at[pl.ds(s * LTOT, LTOT)])
    plsc.subcore_barrier()

    for u in range(NS):
        pltpu.sync_copy(
            sh.at[pl.ds(u * LTOT + s * LCAP, LCAP)],
            lists_v.at[pl.ds(u * LCAP, LCAP)],
        )
    zero16f = jnp.zeros((16,), jnp.float32)

    def zh(i, carry):
        hist_v[pl.ds(i * 16, 16)] = zero16f
        return carry

    lax.fori_loop(0, AROW // 16, zh, 0)
    one16f = jnp.full((16,), 1.0, jnp.float32)

    def hist(i, carry):
        pkv = lists_v[pl.ds(i * 16, 16)]
        rf = jnp.minimum(pkv >> 15, AROW - 1)
        for l in range(16):
            r = rf[l]
            base = r & ~jnp.int32(15)
            vec = jnp.where(iota == (r & 15), one16f, zero16f)
            plsc.addupdate(hist_v.at[pl.ds(base, 16)], vec)
        return carry

    lax.fori_loop(0, LTOT // 16, hist, 0)

    def newt(i, carry):
        x = hist_v[pl.ds(i * 16, 16)] + 1.0
        dis_v[pl.ds(i * 16, 16)] = _nrsqrt(x)
        return carry

    lax.fori_loop(0, AROW // 16, newt, 0)
    pltpu.sync_copy(dis_v.at[pl.ds(16, RNG)],
                    dis_hbm.at[pl.ds(c * NPAD + s * RNG, RNG)])


_scan_kernel = pl.kernel(
    _scan_body,
    out_type=(
        jax.ShapeDtypeStruct((NC * NS * LTOT,), jnp.int32),
        jax.ShapeDtypeStruct((NC * NPAD,), jnp.float32),
    ),
    mesh=_mesh,
    scratch_types=[
        pltpu.VMEM_SHARED((NS * LTOT,), jnp.int32),
        pltpu.VMEM((EPT,), jnp.int32),
        pltpu.VMEM((EPT,), jnp.int32),
        pltpu.VMEM((LTOT,), jnp.int32),
        pltpu.VMEM((AROW,), jnp.float32),
        pltpu.VMEM((AROW,), jnp.float32),
        pltpu.SMEM((NS,), jnp.int32),
    ],
)


def _mm_body(x_ref, w_ref, dis_ref, o_ref):
    xw = jnp.dot(x_ref[0], w_ref[0], preferred_element_type=jnp.float32)
    o_ref[0] = xw * dis_ref[0]


_MM_BLK = 640
_mm = pl.pallas_call(
    _mm_body,
    grid=(NC, NPAD // _MM_BLK),
    in_specs=[
        pl.BlockSpec((1, _MM_BLK, D), lambda c, i: (c, i, 0)),
        pl.BlockSpec((1, D, D), lambda c, i: (c, 0, 0)),
        pl.BlockSpec((1, _MM_BLK, 1), lambda c, i: (c, i, 0)),
    ],
    out_specs=pl.BlockSpec((1, _MM_BLK, D), lambda c, i: (c, i, 0)),
    out_shape=jax.ShapeDtypeStruct((NC, NPAD, D), jnp.float32),
)


def _agg_body(xws_hbm, lists_hbm, dis_hbm, b_hbm, out_hbm,
              acc, pk_v, rows0, rows1, rows2, idx0, idx1, idx2, dis_b, bb,
              gsem0, gsem1, gsem2):
    c = lax.axis_index("c")
    s = lax.axis_index("s")
    zero16f = jnp.zeros((16,), jnp.float32)

    def za(i, carry):
        for jj in range(D // 16):
            acc[i, pl.ds(jj * 16, 16)] = zero16f
        return carry

    lax.fori_loop(0, AROW, za, 0)
    pltpu.sync_copy(b_hbm.at[pl.ds(c * D, D)], bb)
    pltpu.sync_copy(dis_hbm.at[pl.ds(c * NPAD + s * RNG, RNG)],
                    dis_b.at[pl.ds(0, RNG)])

    def unpack_idx(m, idxb):
        for mm in range(GB // 16):
            pkv = pk_v[pl.ds(m * GB + mm * 16, 16)]
            sg = jnp.minimum(pkv & MASK15, NC * NPAD - 1)
            idxb[pl.ds(mm * 16, 16)] = sg

    def accum(m, rows):
        for mm in range(GB // 16):
            pkv = pk_v[pl.ds(m * GB + mm * 16, 16)]
            rf = jnp.minimum(pkv >> 15, AROW - 1)
            for l in range(16):
                r = rf[l]
                k = mm * 16 + l
                for jj in range(D // 16):
                    sl = pl.ds(jj * 16, 16)
                    plsc.addupdate(acc.at[r, sl], rows[k, sl])
        return None

    def run_chunk(u, carry):
        pltpu.sync_copy(
            lists_hbm.at[pl.ds((c * NS + u) * LTOT + s * LCAP, LCAP)], pk_v)
        unpack_idx(0, idx0)
        pltpu.async_copy(xws_hbm.at[idx0], rows0, gsem0)
        unpack_idx(1, idx1)
        pltpu.async_copy(xws_hbm.at[idx1], rows1, gsem1)

        def trip(i, carry2):
            m = 3 * i
            unpack_idx(m + 2, idx2)
            pltpu.async_copy(xws_hbm.at[idx2], rows2, gsem2)
            pltpu.make_async_copy(xws_hbm.at[idx0], rows0, gsem0).wait()
            accum(m, rows0)

            @pl.when(m + 3 < NBAT)
            def _():
                unpack_idx(m + 3, idx0)
                pltpu.async_copy(xws_hbm.at[idx0], rows0, gsem0)

            pltpu.make_async_copy(xws_hbm.at[idx1], rows1, gsem1).wait()
            accum(m + 1, rows1)

            @pl.when(m + 4 < NBAT)
            def _():
                unpack_idx(m + 4, idx1)
                pltpu.async_copy(xws_hbm.at[idx1], rows1, gsem1)

            pltpu.make_async_copy(xws_hbm.at[idx2], rows2, gsem2).wait()
            accum(m + 2, rows2)
            return carry2

        lax.fori_loop(0, NBAT // 3, trip, 0)
        return carry

    lax.fori_loop(0, NS, run_chunk, 0)

    r0g = c * NPAD + s * RNG

    def fchunk(k, carry):
        pltpu.sync_copy(xws_hbm.at[pl.ds(r0g + k * FB, FB)],
                        rows0.at[pl.ds(0, FB)])

        def frow(r, carry2):
            dsc = dis_b[pl.ds(k * FB + r, 16)]
            dis = dsc[0]
            ar = k * FB + r + 16
            for jj in range(D // 16):
                sl = pl.ds(jj * 16, 16)
                v = dis * (acc[ar, sl] + rows0[r, sl]) + bb[sl]
                rows0[r, sl] = jnp.maximum(v, 0.0)
            return carry2

        lax.fori_loop(0, FB, frow, 0)
        pltpu.sync_copy(rows0.at[pl.ds(0, FB)],
                        out_hbm.at[pl.ds(r0g + k * FB, FB)])
        return carry

    lax.fori_loop(0, RNG // FB, fchunk, 0)


_agg_kernel = pl.kernel(
    _agg_body,
    out_type=jax.ShapeDtypeStruct((NC * NPAD, D), jnp.float32),
    mesh=_mesh,
    scratch_types=[
        pltpu.VMEM((AROW, D), jnp.float32),
        pltpu.VMEM((LCAP,), jnp.int32),
        pltpu.VMEM((GB, D), jnp.float32),
        pltpu.VMEM((GB, D), jnp.float32),
        pltpu.VMEM((GB, D), jnp.float32),
        pltpu.VMEM((GB,), jnp.int32),
        pltpu.VMEM((GB,), jnp.int32),
        pltpu.VMEM((GB,), jnp.int32),
        pltpu.VMEM((RNG + 16,), jnp.float32),
        pltpu.VMEM((D,), jnp.float32),
        pltpu.SemaphoreType.DMA,
        pltpu.SemaphoreType.DMA,
        pltpu.SemaphoreType.DMA,
    ],
)


def kernel(x_one, edge_index_one, x_two, edge_index_two, W1, b1, W2, b2):
    src_all = jnp.concatenate([edge_index_one[0], edge_index_two[0]]).astype(jnp.int32)
    dst_all = jnp.concatenate([edge_index_one[1], edge_index_two[1]]).astype(jnp.int32)

    lists, dis = _scan_kernel(src_all, dst_all)

    x_pad = jnp.pad(jnp.stack([x_one, x_two]), ((0, 0), (0, NPAD - N), (0, 0)))
    W_all = jnp.stack([W1, W2])
    xws = _mm(x_pad, W_all, dis.reshape(NC, NPAD, 1))

    b_all = jnp.concatenate([b1, b2])
    out_flat = _agg_kernel(xws.reshape(NC * NPAD, D), lists, dis, b_all)
    out = out_flat.reshape(NC, NPAD, D)
    return (out[0, :N], out[1, :N])
